# baseline (speedup 1.0000x reference)
import functools
import math

import jax
import jax.numpy as jnp
from jax.experimental import pallas as pl
from jax.experimental.pallas import tpu as pltpu

_F32 = jnp.float32
_DEF = jax.lax.Precision.DEFAULT


def _pick_bm(m, target):
    if m % target == 0:
        return target
    return math.gcd(m, target)


def _chunked_dot(x, w, dims, kc=256):
    (xk,), (wk,) = dims[0]
    k = x.shape[xk]
    if k <= kc:
        return jax.lax.dot_general(x, w, dims, preferred_element_type=_F32,
                                   precision=_DEF)
    def sl(a, axis, j):
        idx = [slice(None)] * a.ndim
        idx[axis] = slice(j * kc, (j + 1) * kc)
        return a[tuple(idx)]
    acc = None
    for j in range(k // kc):
        p = jax.lax.dot_general(sl(x, xk, j), sl(w, wk, j), dims,
                                preferred_element_type=_F32, precision=_DEF)
        acc = p if acc is None else acc + p
    return acc


def _extract_patches(x, k, s, p):
    b, h, w, c = x.shape
    ho = (h + 2 * p - k) // s + 1
    wo = (w + 2 * p - k) // s + 1
    xp = jnp.pad(x, ((0, 0), (p, p), (p, p), (0, 0)))
    parts = []
    for ky in range(k):
        for kx in range(k):
            parts.append(xp[:, ky:ky + s * (ho - 1) + 1:s,
                            kx:kx + s * (wo - 1) + 1:s, :])
    pat = jnp.stack(parts, axis=3)
    return pat.reshape(b * ho * wo, k * k * c), ho


def _mm_kernel(x_ref, w_ref, b_ref, o_ref, *, act):
    y = _chunked_dot(x_ref[...], w_ref[...], (((1,), (0,)), ((), ())))
    y = y + b_ref[...]
    if act == 'relu':
        y = jnp.maximum(y, 0.0)
    o_ref[...] = y


def _matmul_bias_act(x, w, b, act, bm_target=512):
    m, k = x.shape
    n = w.shape[1]
    bm = _pick_bm(m, bm_target)
    return pl.pallas_call(
        functools.partial(_mm_kernel, act=act),
        grid=(m // bm,),
        in_specs=[
            pl.BlockSpec((bm, k), lambda i: (i, 0)),
            pl.BlockSpec((k, n), lambda i: (0, 0)),
            pl.BlockSpec((1, n), lambda i: (0, 0)),
        ],
        out_specs=pl.BlockSpec((bm, n), lambda i: (i, 0)),
        out_shape=jax.ShapeDtypeStruct((m, n), _F32),
        compiler_params=pltpu.CompilerParams(
            dimension_semantics=("parallel",)),
    )(x, w, b.reshape(1, n))


def _conv(x, w, b, stride, pad, act):
    o, c, kh, kw = w.shape
    pat, ho = _extract_patches(x, kh, stride, pad)
    wm = w.transpose(2, 3, 1, 0).reshape(kh * kw * c, o)
    y = _matmul_bias_act(pat, wm, b, act)
    return y.reshape(x.shape[0], ho, ho, o)


def _quant_kernel(emb_ref, cb_ref, esq_ref, csq_ref, ids_ref, q_ref,
                  best_ref, bidx_ref, qacc_ref, *, chunk):
    c = pl.program_id(1)
    n_ck = pl.num_programs(1)
    e = emb_ref[...]
    cbc = cb_ref[...]
    p = jax.lax.dot_general(e, cbc, (((1,), (1,)), ((), ())),
                            preferred_element_type=_F32,
                            precision=_DEF)
    dist = (esq_ref[...] + csq_ref[...]) - 2.0 * p
    mv = jnp.min(dist, axis=1, keepdims=True)
    col = jax.lax.broadcasted_iota(jnp.int32, dist.shape, 1)
    mi_loc = jnp.min(jnp.where(dist <= mv, col, chunk), axis=1, keepdims=True)

    take = jnp.logical_or(c == 0, mv < best_ref[...])
    best_ref[...] = jnp.where(take, mv, best_ref[...])
    bidx_ref[...] = jnp.where(take, mi_loc + c * chunk, bidx_ref[...])
    oh = (col == mi_loc).astype(_F32)
    rows = jax.lax.dot_general(oh, cbc, (((1,), (0,)), ((), ())),
                               preferred_element_type=_F32, precision=_DEF)
    qacc_ref[...] = jnp.where(take, rows, qacc_ref[...])

    @pl.when(c == n_ck - 1)
    def _():
        ids_ref[...] = bidx_ref[...]
        q_ref[...] = qacc_ref[...]


def _quantize(emb, codebook, emb_sq, cb_sq, bm_target=784, chunk=512):
    t, d = emb.shape
    n_cb = codebook.shape[0]
    bm = _pick_bm(t, bm_target)
    chunk = min(chunk, n_cb)
    nt = t // bm
    n_ck = n_cb // chunk
    ids3, q = pl.pallas_call(
        functools.partial(_quant_kernel, chunk=chunk),
        grid=(nt, n_ck),
        in_specs=[
            pl.BlockSpec((bm, d), lambda i, c: (i, 0)),
            pl.BlockSpec((chunk, d), lambda i, c: (c, 0)),
            pl.BlockSpec((bm, 1), lambda i, c: (i, 0)),
            pl.BlockSpec((1, chunk), lambda i, c: (0, c)),
        ],
        out_specs=[
            pl.BlockSpec((bm, 1), lambda i, c: (i, 0)),
            pl.BlockSpec((bm, d), lambda i, c: (i, 0)),
        ],
        out_shape=[
            jax.ShapeDtypeStruct((t, 1), jnp.int32),
            jax.ShapeDtypeStruct((t, d), _F32),
        ],
        scratch_shapes=[
            pltpu.VMEM((bm, 1), _F32),
            pltpu.VMEM((bm, 1), jnp.int32),
            pltpu.VMEM((bm, d), _F32),
        ],
        compiler_params=pltpu.CompilerParams(
            dimension_semantics=("parallel", "arbitrary")),
    )(emb, codebook, emb_sq, cb_sq)
    return ids3.reshape(t), q


def _mlp_kernel(q_ref, fc1w_ref, fc1b_ref, fc2w_ref, fc2b_ref, g_ref, b_ref,
                o_ref, acc_ref, *, n_h):
    hstep = pl.program_id(1)
    q = q_ref[...]
    hid = jax.lax.dot_general(q, fc1w_ref[...], (((1,), (1,)), ((), ())),
                              preferred_element_type=_F32, precision=_DEF)
    hid = hid + fc1b_ref[...]
    hid = 0.5 * hid * (1.0 + jax.lax.erf(hid * (1.0 / math.sqrt(2.0))))
    contrib = _chunked_dot(hid, fc2w_ref[...], (((1,), (1,)), ((), ())))

    @pl.when(hstep == 0)
    def _():
        acc_ref[...] = contrib

    @pl.when(hstep > 0)
    def _():
        acc_ref[...] = acc_ref[...] + contrib

    @pl.when(hstep == n_h - 1)
    def _():
        y = acc_ref[...] + fc2b_ref[...]
        mu = jnp.mean(y, axis=1, keepdims=True)
        yc = y - mu
        var = jnp.mean(yc * yc, axis=1, keepdims=True)
        o_ref[...] = yc * jax.lax.rsqrt(var + 1e-5) * g_ref[...] + b_ref[...]


def _mlp_ln(q, fc1_w, fc1_b, fc2_w, fc2_b, ln_g, ln_b,
            bm_target=784, bh_target=512):
    t, d = q.shape
    hdim, _ = fc1_w.shape
    ldim = fc2_w.shape[0]
    bm = _pick_bm(t, bm_target)
    bh = _pick_bm(hdim, bh_target)
    n_h = hdim // bh
    return pl.pallas_call(
        functools.partial(_mlp_kernel, n_h=n_h),
        grid=(t // bm, n_h),
        in_specs=[
            pl.BlockSpec((bm, d), lambda i, h: (i, 0)),
            pl.BlockSpec((bh, d), lambda i, h: (h, 0)),
            pl.BlockSpec((1, bh), lambda i, h: (0, h)),
            pl.BlockSpec((ldim, bh), lambda i, h: (0, h)),
            pl.BlockSpec((1, ldim), lambda i, h: (0, 0)),
            pl.BlockSpec((1, ldim), lambda i, h: (0, 0)),
            pl.BlockSpec((1, ldim), lambda i, h: (0, 0)),
        ],
        out_specs=pl.BlockSpec((bm, ldim), lambda i, h: (i, 0)),
        out_shape=jax.ShapeDtypeStruct((t, ldim), _F32),
        scratch_shapes=[pltpu.VMEM((bm, ldim), _F32)],
        compiler_params=pltpu.CompilerParams(
            dimension_semantics=("parallel", "arbitrary")),
    )(q, fc1_w, fc1_b.reshape(1, hdim), fc2_w, fc2_b.reshape(1, ldim),
      ln_g.reshape(1, ldim), ln_b.reshape(1, ldim))


def kernel(img_fused, w1, b1, w2, b2, w3, b3, w4, b4, codebook,
           fc1_w, fc1_b, fc2_w, fc2_b, ln_g, ln_b):
    bsz = img_fused.shape[0]
    x = img_fused.transpose(0, 2, 3, 1)
    h = _conv(x, w1, b1, 2, 1, 'relu')
    h = _conv(h, w2, b2, 2, 1, 'relu')
    h = _conv(h, w3, b3, 2, 1, 'relu')
    feat = _conv(h, w4, b4, 1, 1, 'none')
    d = feat.shape[-1]
    emb = feat.reshape(bsz * feat.shape[1] * feat.shape[2], d)
    emb_sq = jnp.sum(emb ** 2, axis=-1).reshape(-1, 1)
    cb_sq = jnp.sum(codebook ** 2, axis=-1).reshape(1, -1)
    ids, q = _quantize(emb, codebook, emb_sq, cb_sq)
    out = _mlp_ln(q, fc1_w, fc1_b, fc2_w, fc2_b, ln_g, ln_b)
    n = emb.shape[0] // bsz
    return out.reshape(bsz, n, fc2_w.shape[0]), ids.reshape(bsz, n)


# speedup vs baseline: 3.4424x; 3.4424x over previous
import functools
import math

import jax
import jax.numpy as jnp
from jax.experimental import pallas as pl
from jax.experimental.pallas import tpu as pltpu

_F32 = jnp.float32
_DEF = jax.lax.Precision.DEFAULT


def _pick_bm(m, target):
    if m % target == 0:
        return target
    return math.gcd(m, target)


def _chunked_dot(x, w, dims, kc=256):
    (xk,), (wk,) = dims[0]
    k = x.shape[xk]
    if k <= kc:
        return jax.lax.dot_general(x, w, dims, preferred_element_type=_F32,
                                   precision=_DEF)
    def sl(a, axis, j):
        idx = [slice(None)] * a.ndim
        idx[axis] = slice(j * kc, (j + 1) * kc)
        return a[tuple(idx)]
    acc = None
    for j in range(k // kc):
        p = jax.lax.dot_general(sl(x, xk, j), sl(w, wk, j), dims,
                                preferred_element_type=_F32, precision=_DEF)
        acc = p if acc is None else acc + p
    return acc


def _extract_patches(x, k, s, p):
    b, h, w, c = x.shape
    ho = (h + 2 * p - k) // s + 1
    wo = (w + 2 * p - k) // s + 1
    xp = jnp.pad(x, ((0, 0), (p, p), (p, p), (0, 0)))
    parts = []
    for ky in range(k):
        for kx in range(k):
            parts.append(xp[:, ky:ky + s * (ho - 1) + 1:s,
                            kx:kx + s * (wo - 1) + 1:s, :])
    pat = jnp.stack(parts, axis=3)
    return pat.reshape(b * ho * wo, k * k * c), ho


def _mm_kernel(x_ref, w_ref, b_ref, o_ref, *, act):
    y = _chunked_dot(x_ref[...], w_ref[...], (((1,), (0,)), ((), ())))
    y = y + b_ref[...]
    if act == 'relu':
        y = jnp.maximum(y, 0.0)
    o_ref[...] = y


def _matmul_bias_act(x, w, b, act, bm_target=512):
    m, k = x.shape
    n = w.shape[1]
    bm = _pick_bm(m, bm_target)
    return pl.pallas_call(
        functools.partial(_mm_kernel, act=act),
        grid=(m // bm,),
        in_specs=[
            pl.BlockSpec((bm, k), lambda i: (i, 0)),
            pl.BlockSpec((k, n), lambda i: (0, 0)),
            pl.BlockSpec((1, n), lambda i: (0, 0)),
        ],
        out_specs=pl.BlockSpec((bm, n), lambda i: (i, 0)),
        out_shape=jax.ShapeDtypeStruct((m, n), _F32),
        compiler_params=pltpu.CompilerParams(
            dimension_semantics=("parallel",)),
    )(x, w, b.reshape(1, n))


def _conv(x, w, b, stride, pad, act):
    o, c, kh, kw = w.shape
    pat, ho = _extract_patches(x, kh, stride, pad)
    wm = w.transpose(2, 3, 1, 0).reshape(kh * kw * c, o)
    y = _matmul_bias_act(pat, wm, b, act, bm_target=7168)
    return y.reshape(x.shape[0], ho, ho, o)


def _conv_s2_kernel(xa_ref, xb_ref, xc_ref, xd_ref, w_ref, b_ref, o_ref,
                    *, cin, ho, act):
    xs = {(0, 0): xa_ref, (0, 1): xb_ref, (1, 0): xc_ref, (1, 1): xd_ref}
    gtaps = max(1, 256 // cin)
    acc = None
    for g in range(16 // gtaps):
        ops = []
        for t in range(g * gtaps, (g + 1) * gtaps):
            ky, kx = t // 4, t % 4
            ref = xs[(ky % 2, kx % 2)]
            sl = ref[0, pl.ds(ky // 2, ho), pl.ds(kx // 2, ho), :]
            ops.append(sl.reshape(ho * ho, cin))
        opnd = ops[0] if len(ops) == 1 else jnp.concatenate(ops, axis=1)
        p = jax.lax.dot_general(
            opnd, w_ref[pl.ds(g * gtaps * cin, gtaps * cin), :],
            (((1,), (0,)), ((), ())),
            preferred_element_type=_F32, precision=_DEF)
        acc = p if acc is None else acc + p
    y = acc + b_ref[...]
    if act == 'relu':
        y = jnp.maximum(y, 0.0)
    o_ref[0] = y


def _conv_s2_fused(x, w, b, act):
    bsz, h, _, cin = x.shape
    cout = w.shape[0]
    ho = h // 2
    hp = (h + 2) // 2
    xp = jnp.pad(x, ((0, 0), (1, 1), (1, 1), (0, 0)))
    xv = xp.reshape(bsz, hp, 2, hp, 2, cin).transpose(0, 2, 4, 1, 3, 5)
    planes = [xv[:, qy, qx] for qy in (0, 1) for qx in (0, 1)]
    wm = w.transpose(2, 3, 1, 0).reshape(16 * cin, cout)
    out = pl.pallas_call(
        functools.partial(_conv_s2_kernel, cin=cin, ho=ho, act=act),
        grid=(bsz,),
        in_specs=[pl.BlockSpec((1, hp, hp, cin), lambda i: (i, 0, 0, 0))
                  for _ in range(4)] + [
            pl.BlockSpec((16 * cin, cout), lambda i: (0, 0)),
            pl.BlockSpec((1, cout), lambda i: (0, 0)),
        ],
        out_specs=pl.BlockSpec((1, ho * ho, cout), lambda i: (i, 0, 0)),
        out_shape=jax.ShapeDtypeStruct((bsz, ho * ho, cout), _F32),
        compiler_params=pltpu.CompilerParams(
            dimension_semantics=("parallel",)),
    )(*planes, wm, b.reshape(1, cout))
    return out.reshape(bsz, ho, ho, cout)


def _conv_s1_kernel(x_ref, w_ref, b_ref, o_ref, *, cin, ho):
    acc = None
    for t in range(9):
        ky, kx = t // 3, t % 3
        sl = x_ref[0, pl.ds(ky, ho), pl.ds(kx, ho), :]
        p = jax.lax.dot_general(
            sl.reshape(ho * ho, cin), w_ref[pl.ds(t * cin, cin), :],
            (((1,), (0,)), ((), ())),
            preferred_element_type=_F32, precision=_DEF)
        acc = p if acc is None else acc + p
    o_ref[0] = acc + b_ref[...]


def _conv_s1_fused(x, w, b):
    bsz, h, _, cin = x.shape
    cout = w.shape[0]
    xp = jnp.pad(x, ((0, 0), (1, 1), (1, 1), (0, 0)))
    wm = w.transpose(2, 3, 1, 0).reshape(9 * cin, cout)
    out = pl.pallas_call(
        functools.partial(_conv_s1_kernel, cin=cin, ho=h),
        grid=(bsz,),
        in_specs=[
            pl.BlockSpec((1, h + 2, h + 2, cin), lambda i: (i, 0, 0, 0)),
            pl.BlockSpec((9 * cin, cout), lambda i: (0, 0)),
            pl.BlockSpec((1, cout), lambda i: (0, 0)),
        ],
        out_specs=pl.BlockSpec((1, h * h, cout), lambda i: (i, 0, 0)),
        out_shape=jax.ShapeDtypeStruct((bsz, h * h, cout), _F32),
        compiler_params=pltpu.CompilerParams(
            dimension_semantics=("parallel",)),
    )(xp, wm, b.reshape(1, cout))
    return out.reshape(bsz, h, h, cout)


def _quant_kernel(emb_ref, cb_ref, esq_ref, csq_ref, ids_ref, q_ref,
                  best_ref, bidx_ref, qacc_ref, *, chunk):
    c = pl.program_id(1)
    n_ck = pl.num_programs(1)
    e = emb_ref[...]
    cbc = cb_ref[...]
    p = jax.lax.dot_general(e, cbc, (((1,), (1,)), ((), ())),
                            preferred_element_type=_F32,
                            precision=_DEF)
    dist = (esq_ref[...] + csq_ref[...]) - 2.0 * p
    mv = jnp.min(dist, axis=1, keepdims=True)
    col = jax.lax.broadcasted_iota(jnp.int32, dist.shape, 1)
    mi_loc = jnp.min(jnp.where(dist <= mv, col, chunk), axis=1, keepdims=True)

    take = jnp.logical_or(c == 0, mv < best_ref[...])
    best_ref[...] = jnp.where(take, mv, best_ref[...])
    bidx_ref[...] = jnp.where(take, mi_loc + c * chunk, bidx_ref[...])
    oh = (col == mi_loc).astype(_F32)
    rows = jax.lax.dot_general(oh, cbc, (((1,), (0,)), ((), ())),
                               preferred_element_type=_F32, precision=_DEF)
    qacc_ref[...] = jnp.where(take, rows, qacc_ref[...])

    @pl.when(c == n_ck - 1)
    def _():
        ids_ref[...] = bidx_ref[...]
        q_ref[...] = qacc_ref[...]


def _quantize(emb, codebook, emb_sq, cb_sq, bm_target=784, chunk=512):
    t, d = emb.shape
    n_cb = codebook.shape[0]
    bm = _pick_bm(t, bm_target)
    chunk = min(chunk, n_cb)
    nt = t // bm
    n_ck = n_cb // chunk
    ids3, q = pl.pallas_call(
        functools.partial(_quant_kernel, chunk=chunk),
        grid=(nt, n_ck),
        in_specs=[
            pl.BlockSpec((bm, d), lambda i, c: (i, 0)),
            pl.BlockSpec((chunk, d), lambda i, c: (c, 0)),
            pl.BlockSpec((bm, 1), lambda i, c: (i, 0)),
            pl.BlockSpec((1, chunk), lambda i, c: (0, c)),
        ],
        out_specs=[
            pl.BlockSpec((bm, 1), lambda i, c: (i, 0)),
            pl.BlockSpec((bm, d), lambda i, c: (i, 0)),
        ],
        out_shape=[
            jax.ShapeDtypeStruct((t, 1), jnp.int32),
            jax.ShapeDtypeStruct((t, d), _F32),
        ],
        scratch_shapes=[
            pltpu.VMEM((bm, 1), _F32),
            pltpu.VMEM((bm, 1), jnp.int32),
            pltpu.VMEM((bm, d), _F32),
        ],
        compiler_params=pltpu.CompilerParams(
            dimension_semantics=("parallel", "arbitrary")),
    )(emb, codebook, emb_sq, cb_sq)
    return ids3.reshape(t), q


def _mlp_kernel(q_ref, fc1w_ref, fc1b_ref, fc2w_ref, fc2b_ref, g_ref, b_ref,
                o_ref, acc_ref, *, n_h):
    hstep = pl.program_id(1)
    q = q_ref[...]
    hid = jax.lax.dot_general(q, fc1w_ref[...], (((1,), (1,)), ((), ())),
                              preferred_element_type=_F32, precision=_DEF)
    hid = hid + fc1b_ref[...]
    hid = 0.5 * hid * (1.0 + jax.lax.erf(hid * (1.0 / math.sqrt(2.0))))
    contrib = _chunked_dot(hid, fc2w_ref[...], (((1,), (1,)), ((), ())))

    @pl.when(hstep == 0)
    def _():
        acc_ref[...] = contrib

    @pl.when(hstep > 0)
    def _():
        acc_ref[...] = acc_ref[...] + contrib

    @pl.when(hstep == n_h - 1)
    def _():
        y = acc_ref[...] + fc2b_ref[...]
        mu = jnp.mean(y, axis=1, keepdims=True)
        yc = y - mu
        var = jnp.mean(yc * yc, axis=1, keepdims=True)
        o_ref[...] = yc * jax.lax.rsqrt(var + 1e-5) * g_ref[...] + b_ref[...]


def _mlp_ln(q, fc1_w, fc1_b, fc2_w, fc2_b, ln_g, ln_b,
            bm_target=784, bh_target=512):
    t, d = q.shape
    hdim, _ = fc1_w.shape
    ldim = fc2_w.shape[0]
    bm = _pick_bm(t, bm_target)
    bh = _pick_bm(hdim, bh_target)
    n_h = hdim // bh
    return pl.pallas_call(
        functools.partial(_mlp_kernel, n_h=n_h),
        grid=(t // bm, n_h),
        in_specs=[
            pl.BlockSpec((bm, d), lambda i, h: (i, 0)),
            pl.BlockSpec((bh, d), lambda i, h: (h, 0)),
            pl.BlockSpec((1, bh), lambda i, h: (0, h)),
            pl.BlockSpec((ldim, bh), lambda i, h: (0, h)),
            pl.BlockSpec((1, ldim), lambda i, h: (0, 0)),
            pl.BlockSpec((1, ldim), lambda i, h: (0, 0)),
            pl.BlockSpec((1, ldim), lambda i, h: (0, 0)),
        ],
        out_specs=pl.BlockSpec((bm, ldim), lambda i, h: (i, 0)),
        out_shape=jax.ShapeDtypeStruct((t, ldim), _F32),
        scratch_shapes=[pltpu.VMEM((bm, ldim), _F32)],
        compiler_params=pltpu.CompilerParams(
            dimension_semantics=("parallel", "arbitrary")),
    )(q, fc1_w, fc1_b.reshape(1, hdim), fc2_w, fc2_b.reshape(1, ldim),
      ln_g.reshape(1, ldim), ln_b.reshape(1, ldim))


def kernel(img_fused, w1, b1, w2, b2, w3, b3, w4, b4, codebook,
           fc1_w, fc1_b, fc2_w, fc2_b, ln_g, ln_b):
    bsz = img_fused.shape[0]
    x = img_fused.transpose(0, 2, 3, 1)
    h = _conv(x, w1, b1, 2, 1, 'relu')
    h = _conv_s2_fused(h, w2, b2, 'relu')
    h = _conv_s2_fused(h, w3, b3, 'relu')
    feat = _conv_s1_fused(h, w4, b4)
    d = feat.shape[-1]
    emb = feat.reshape(bsz * feat.shape[1] * feat.shape[2], d)
    emb_sq = jnp.sum(emb ** 2, axis=-1).reshape(-1, 1)
    cb_sq = jnp.sum(codebook ** 2, axis=-1).reshape(1, -1)
    ids, q = _quantize(emb, codebook, emb_sq, cb_sq)
    out = _mlp_ln(q, fc1_w, fc1_b, fc2_w, fc2_b, ln_g, ln_b)
    n = emb.shape[0] // bsz
    return out.reshape(bsz, n, fc2_w.shape[0]), ids.reshape(bsz, n)


# speedup vs baseline: 3.5153x; 1.0212x over previous
import functools
import math

import jax
import jax.numpy as jnp
from jax.experimental import pallas as pl
from jax.experimental.pallas import tpu as pltpu
from jax.experimental.pallas import tpu_sc as plsc

_F32 = jnp.float32
_DEF = jax.lax.Precision.DEFAULT


def _pick_bm(m, target):
    if m % target == 0:
        return target
    return math.gcd(m, target)


def _chunked_dot(x, w, dims, kc=256):
    (xk,), (wk,) = dims[0]
    k = x.shape[xk]
    if k <= kc:
        return jax.lax.dot_general(x, w, dims, preferred_element_type=_F32,
                                   precision=_DEF)
    def sl(a, axis, j):
        idx = [slice(None)] * a.ndim
        idx[axis] = slice(j * kc, (j + 1) * kc)
        return a[tuple(idx)]
    acc = None
    for j in range(k // kc):
        p = jax.lax.dot_general(sl(x, xk, j), sl(w, wk, j), dims,
                                preferred_element_type=_F32, precision=_DEF)
        acc = p if acc is None else acc + p
    return acc


def _extract_patches(x, k, s, p):
    b, h, w, c = x.shape
    ho = (h + 2 * p - k) // s + 1
    wo = (w + 2 * p - k) // s + 1
    xp = jnp.pad(x, ((0, 0), (p, p), (p, p), (0, 0)))
    parts = []
    for ky in range(k):
        for kx in range(k):
            parts.append(xp[:, ky:ky + s * (ho - 1) + 1:s,
                            kx:kx + s * (wo - 1) + 1:s, :])
    pat = jnp.stack(parts, axis=3)
    return pat.reshape(b * ho * wo, k * k * c), ho


def _mm_kernel(x_ref, w_ref, b_ref, o_ref, *, act):
    y = _chunked_dot(x_ref[...], w_ref[...], (((1,), (0,)), ((), ())))
    y = y + b_ref[...]
    if act == 'relu':
        y = jnp.maximum(y, 0.0)
    o_ref[...] = y


def _matmul_bias_act(x, w, b, act, bm_target=512):
    m, k = x.shape
    n = w.shape[1]
    bm = _pick_bm(m, bm_target)
    return pl.pallas_call(
        functools.partial(_mm_kernel, act=act),
        grid=(m // bm,),
        in_specs=[
            pl.BlockSpec((bm, k), lambda i: (i, 0)),
            pl.BlockSpec((k, n), lambda i: (0, 0)),
            pl.BlockSpec((1, n), lambda i: (0, 0)),
        ],
        out_specs=pl.BlockSpec((bm, n), lambda i: (i, 0)),
        out_shape=jax.ShapeDtypeStruct((m, n), _F32),
        compiler_params=pltpu.CompilerParams(
            dimension_semantics=("parallel",)),
    )(x, w, b.reshape(1, n))


def _conv(x, w, b, stride, pad, act):
    o, c, kh, kw = w.shape
    pat, ho = _extract_patches(x, kh, stride, pad)
    wm = w.transpose(2, 3, 1, 0).reshape(kh * kw * c, o)
    y = _matmul_bias_act(pat, wm, b, act, bm_target=7168)
    return y.reshape(x.shape[0], ho, ho, o)


def _conv_s2_kernel(xa_ref, xb_ref, xc_ref, xd_ref, w_ref, b_ref, o_ref,
                    *, cin, ho, act):
    xs = {(0, 0): xa_ref, (0, 1): xb_ref, (1, 0): xc_ref, (1, 1): xd_ref}
    gtaps = max(1, 256 // cin)
    acc = None
    for g in range(16 // gtaps):
        ops = []
        for t in range(g * gtaps, (g + 1) * gtaps):
            ky, kx = t // 4, t % 4
            ref = xs[(ky % 2, kx % 2)]
            sl = ref[0, pl.ds(ky // 2, ho), pl.ds(kx // 2, ho), :]
            ops.append(sl.reshape(ho * ho, cin))
        opnd = ops[0] if len(ops) == 1 else jnp.concatenate(ops, axis=1)
        p = jax.lax.dot_general(
            opnd, w_ref[pl.ds(g * gtaps * cin, gtaps * cin), :],
            (((1,), (0,)), ((), ())),
            preferred_element_type=_F32, precision=_DEF)
        acc = p if acc is None else acc + p
    y = acc + b_ref[...]
    if act == 'relu':
        y = jnp.maximum(y, 0.0)
    o_ref[0] = y


def _conv_s2_fused(x, w, b, act):
    bsz, h, _, cin = x.shape
    cout = w.shape[0]
    ho = h // 2
    hp = (h + 2) // 2
    xp = jnp.pad(x, ((0, 0), (1, 1), (1, 1), (0, 0)))
    xv = xp.reshape(bsz, hp, 2, hp, 2, cin).transpose(0, 2, 4, 1, 3, 5)
    planes = [xv[:, qy, qx] for qy in (0, 1) for qx in (0, 1)]
    wm = w.transpose(2, 3, 1, 0).reshape(16 * cin, cout)
    out = pl.pallas_call(
        functools.partial(_conv_s2_kernel, cin=cin, ho=ho, act=act),
        grid=(bsz,),
        in_specs=[pl.BlockSpec((1, hp, hp, cin), lambda i: (i, 0, 0, 0))
                  for _ in range(4)] + [
            pl.BlockSpec((16 * cin, cout), lambda i: (0, 0)),
            pl.BlockSpec((1, cout), lambda i: (0, 0)),
        ],
        out_specs=pl.BlockSpec((1, ho * ho, cout), lambda i: (i, 0, 0)),
        out_shape=jax.ShapeDtypeStruct((bsz, ho * ho, cout), _F32),
        compiler_params=pltpu.CompilerParams(
            dimension_semantics=("parallel",)),
    )(*planes, wm, b.reshape(1, cout))
    return out.reshape(bsz, ho, ho, cout)


def _conv_s1_kernel(x_ref, w_ref, b_ref, o_ref, *, cin, ho):
    acc = None
    for t in range(9):
        ky, kx = t // 3, t % 3
        sl = x_ref[0, pl.ds(ky, ho), pl.ds(kx, ho), :]
        p = jax.lax.dot_general(
            sl.reshape(ho * ho, cin), w_ref[pl.ds(t * cin, cin), :],
            (((1,), (0,)), ((), ())),
            preferred_element_type=_F32, precision=_DEF)
        acc = p if acc is None else acc + p
    o_ref[0] = acc + b_ref[...]


def _conv_s1_fused(x, w, b):
    bsz, h, _, cin = x.shape
    cout = w.shape[0]
    xp = jnp.pad(x, ((0, 0), (1, 1), (1, 1), (0, 0)))
    wm = w.transpose(2, 3, 1, 0).reshape(9 * cin, cout)
    out = pl.pallas_call(
        functools.partial(_conv_s1_kernel, cin=cin, ho=h),
        grid=(bsz,),
        in_specs=[
            pl.BlockSpec((1, h + 2, h + 2, cin), lambda i: (i, 0, 0, 0)),
            pl.BlockSpec((9 * cin, cout), lambda i: (0, 0)),
            pl.BlockSpec((1, cout), lambda i: (0, 0)),
        ],
        out_specs=pl.BlockSpec((1, h * h, cout), lambda i: (i, 0, 0)),
        out_shape=jax.ShapeDtypeStruct((bsz, h * h, cout), _F32),
        compiler_params=pltpu.CompilerParams(
            dimension_semantics=("parallel",)),
    )(xp, wm, b.reshape(1, cout))
    return out.reshape(bsz, h, h, cout)


def _quant_kernel(emb_ref, cb_ref, esq_ref, csq_ref, ids_ref,
                  best_ref, bidx_ref, *, chunk):
    c = pl.program_id(1)
    n_ck = pl.num_programs(1)
    e = emb_ref[...]
    cbc = cb_ref[...]
    p = jax.lax.dot_general(e, cbc, (((1,), (1,)), ((), ())),
                            preferred_element_type=_F32,
                            precision=_DEF)
    dist = (esq_ref[...] + csq_ref[...]) - 2.0 * p
    mv = jnp.min(dist, axis=1, keepdims=True)
    col = jax.lax.broadcasted_iota(jnp.int32, dist.shape, 1)
    mi_loc = jnp.min(jnp.where(dist <= mv, col, chunk), axis=1, keepdims=True)

    take = jnp.logical_or(c == 0, mv < best_ref[...])
    best_ref[...] = jnp.where(take, mv, best_ref[...])
    bidx_ref[...] = jnp.where(take, mi_loc + c * chunk, bidx_ref[...])

    @pl.when(c == n_ck - 1)
    def _():
        ids_ref[...] = bidx_ref[...]


def _quantize(emb, codebook, emb_sq, cb_sq, bm_target=784, chunk=1024):
    t, d = emb.shape
    n_cb = codebook.shape[0]
    bm = _pick_bm(t, bm_target)
    chunk = min(chunk, n_cb)
    nt = t // bm
    n_ck = n_cb // chunk
    ids3 = pl.pallas_call(
        functools.partial(_quant_kernel, chunk=chunk),
        grid=(nt, n_ck),
        in_specs=[
            pl.BlockSpec((bm, d), lambda i, c: (i, 0)),
            pl.BlockSpec((chunk, d), lambda i, c: (c, 0)),
            pl.BlockSpec((bm, 1), lambda i, c: (i, 0)),
            pl.BlockSpec((1, chunk), lambda i, c: (0, c)),
        ],
        out_specs=pl.BlockSpec((bm, 1), lambda i, c: (i, 0)),
        out_shape=jax.ShapeDtypeStruct((t, 1), jnp.int32),
        scratch_shapes=[
            pltpu.VMEM((bm, 1), _F32),
            pltpu.VMEM((bm, 1), jnp.int32),
        ],
        compiler_params=pltpu.CompilerParams(
            dimension_semantics=("parallel", "arbitrary")),
    )(emb, codebook, emb_sq, cb_sq)
    return ids3.reshape(t)


def _sc_gather(table, idx):
    info = plsc.get_sparse_core_info()
    nw = info.num_cores * info.num_subcores
    b = idx.shape[0]
    d = table.shape[1]
    b_per_w = b // nw
    mesh = plsc.VectorSubcoreMesh(core_axis_name="c", subcore_axis_name="s")

    @functools.partial(
        pl.kernel, mesh=mesh,
        out_type=jax.ShapeDtypeStruct((b, d), _F32),
        scratch_types=[
            pltpu.VMEM((b_per_w,), jnp.int32),
            pltpu.VMEM((b_per_w, d), _F32),
            pltpu.SemaphoreType.DMA,
        ],
    )
    def k(table_hbm, idx_hbm, out_hbm, idx_v, rows_v, sem):
        wid = jax.lax.axis_index("s") * info.num_cores + jax.lax.axis_index("c")
        base = wid * b_per_w
        pltpu.sync_copy(idx_hbm.at[pl.ds(base, b_per_w)], idx_v)
        pltpu.async_copy(table_hbm.at[idx_v], rows_v, sem).wait()
        pltpu.sync_copy(rows_v, out_hbm.at[pl.ds(base, b_per_w)])

    return k(table, idx)


def _mlp_kernel(q_ref, fc1w_ref, fc1b_ref, fc2w_ref, fc2b_ref, g_ref, b_ref,
                o_ref, acc_ref, *, n_h):
    hstep = pl.program_id(1)
    q = q_ref[...]
    hid = jax.lax.dot_general(q, fc1w_ref[...], (((1,), (1,)), ((), ())),
                              preferred_element_type=_F32, precision=_DEF)
    hid = hid + fc1b_ref[...]
    hid = 0.5 * hid * (1.0 + jax.lax.erf(hid * (1.0 / math.sqrt(2.0))))
    contrib = _chunked_dot(hid, fc2w_ref[...], (((1,), (1,)), ((), ())))

    @pl.when(hstep == 0)
    def _():
        acc_ref[...] = contrib

    @pl.when(hstep > 0)
    def _():
        acc_ref[...] = acc_ref[...] + contrib

    @pl.when(hstep == n_h - 1)
    def _():
        y = acc_ref[...] + fc2b_ref[...]
        mu = jnp.mean(y, axis=1, keepdims=True)
        yc = y - mu
        var = jnp.mean(yc * yc, axis=1, keepdims=True)
        o_ref[...] = yc * jax.lax.rsqrt(var + 1e-5) * g_ref[...] + b_ref[...]


def _mlp_ln(q, fc1_w, fc1_b, fc2_w, fc2_b, ln_g, ln_b,
            bm_target=784, bh_target=512):
    t, d = q.shape
    hdim, _ = fc1_w.shape
    ldim = fc2_w.shape[0]
    bm = _pick_bm(t, bm_target)
    bh = _pick_bm(hdim, bh_target)
    n_h = hdim // bh
    return pl.pallas_call(
        functools.partial(_mlp_kernel, n_h=n_h),
        grid=(t // bm, n_h),
        in_specs=[
            pl.BlockSpec((bm, d), lambda i, h: (i, 0)),
            pl.BlockSpec((bh, d), lambda i, h: (h, 0)),
            pl.BlockSpec((1, bh), lambda i, h: (0, h)),
            pl.BlockSpec((ldim, bh), lambda i, h: (0, h)),
            pl.BlockSpec((1, ldim), lambda i, h: (0, 0)),
            pl.BlockSpec((1, ldim), lambda i, h: (0, 0)),
            pl.BlockSpec((1, ldim), lambda i, h: (0, 0)),
        ],
        out_specs=pl.BlockSpec((bm, ldim), lambda i, h: (i, 0)),
        out_shape=jax.ShapeDtypeStruct((t, ldim), _F32),
        scratch_shapes=[pltpu.VMEM((bm, ldim), _F32)],
        compiler_params=pltpu.CompilerParams(
            dimension_semantics=("parallel", "arbitrary")),
    )(q, fc1_w, fc1_b.reshape(1, hdim), fc2_w, fc2_b.reshape(1, ldim),
      ln_g.reshape(1, ldim), ln_b.reshape(1, ldim))


def kernel(img_fused, w1, b1, w2, b2, w3, b3, w4, b4, codebook,
           fc1_w, fc1_b, fc2_w, fc2_b, ln_g, ln_b):
    bsz = img_fused.shape[0]
    x = img_fused.transpose(0, 2, 3, 1)
    h = _conv(x, w1, b1, 2, 1, 'relu')
    h = _conv_s2_fused(h, w2, b2, 'relu')
    h = _conv_s2_fused(h, w3, b3, 'relu')
    feat = _conv_s1_fused(h, w4, b4)
    d = feat.shape[-1]
    emb = feat.reshape(bsz * feat.shape[1] * feat.shape[2], d)
    emb_sq = jnp.sum(emb ** 2, axis=-1).reshape(-1, 1)
    cb_sq = jnp.sum(codebook ** 2, axis=-1).reshape(1, -1)
    ids = _quantize(emb, codebook, emb_sq, cb_sq)
    t = ids.shape[0]
    nw_pad = (-t) % 256
    ids_pad = jnp.concatenate([ids, jnp.zeros((nw_pad,), jnp.int32)]) \
        if nw_pad else ids
    q = _sc_gather(codebook, ids_pad)[:t]
    out = _mlp_ln(q, fc1_w, fc1_b, fc2_w, fc2_b, ln_g, ln_b)
    n = emb.shape[0] // bsz
    return out.reshape(bsz, n, fc2_w.shape[0]), ids.reshape(bsz, n)
